# baseline (speedup 1.0000x reference)
import functools

import jax
import jax.numpy as jnp
from jax import lax
from jax.experimental import pallas as pl
from jax.experimental.pallas import tpu as pltpu
from jax.experimental.pallas import tpu_sc as plsc

_N = 10000
_E = 320000
_G = 16
_CW = 128
_NTILES = 16
_NCORES = 2
_NW = _NCORES * _NTILES
_K = 128
_NCHUNK = 80
_EPT = _NCHUNK * _K
_EPAD = _NW * _EPT
_RPT = 640
_NPAD = _RPT * _NTILES

_PREC = lax.Precision.HIGHEST


def _sc_mesh():
  return plsc.VectorSubcoreMesh(core_axis_name="c", subcore_axis_name="s")


@functools.partial(
    pl.kernel,
    out_type=jax.ShapeDtypeStruct((_NCORES, _NPAD, _CW), jnp.float32),
    mesh=_sc_mesh(),
    scratch_types=[
        pltpu.VMEM((_K,), jnp.int32),
        pltpu.VMEM((_K, _CW), jnp.float32),
        pltpu.VMEM((_K, _CW), jnp.float32),
        pltpu.VMEM_SHARED((_NPAD, _CW), jnp.float32),
    ],
)
def _sc_degree(dst_hbm, out_hbm, di_v, ones_v, zb_v, acc_sh):
  c = lax.axis_index("c")
  s = lax.axis_index("s")
  tid = c * _NTILES + s
  zeros16 = jnp.zeros((16,), jnp.float32)
  ones16 = jnp.ones((16,), jnp.float32)

  def fill(i, carry):
    r = i // (_CW // 16)
    col = (i % (_CW // 16)) * 16
    ones_v[r, pl.ds(col, 16)] = ones16
    zb_v[r, pl.ds(col, 16)] = zeros16
    return carry
  lax.fori_loop(0, _K * _CW // 16, fill, 0)

  for k in range(_RPT // _K):
    pltpu.sync_copy(zb_v, acc_sh.at[pl.ds(s * _RPT + k * _K, _K)])
  plsc.subcore_barrier()

  def chunk(j, carry):
    pltpu.sync_copy(dst_hbm.at[tid, j], di_v)
    pltpu.sync_copy(ones_v, acc_sh.at[di_v], add=True)
    return carry
  lax.fori_loop(0, _NCHUNK, chunk, 0)

  plsc.subcore_barrier()
  pltpu.sync_copy(acc_sh.at[pl.ds(s * _RPT, _RPT)],
                  out_hbm.at[c, pl.ds(s * _RPT, _RPT)])


def _make_sc_spmm(ncb):
  @functools.partial(
      pl.kernel,
      out_type=jax.ShapeDtypeStruct((_NCORES, ncb, _NPAD, _CW), jnp.float32),
      mesh=_sc_mesh(),
      scratch_types=[
          pltpu.VMEM((_K,), jnp.int32),
          pltpu.VMEM((_K,), jnp.int32),
          pltpu.VMEM((_K,), jnp.int32),
          pltpu.VMEM((_K,), jnp.int32),
          pltpu.VMEM((_K, _CW), jnp.float32),
          pltpu.VMEM((_K, _CW), jnp.float32),
          pltpu.VMEM((_K // 2, _CW), jnp.float32),
          pltpu.VMEM_SHARED((_NPAD, _CW), jnp.float32),
          pltpu.SemaphoreType.DMA,
          pltpu.SemaphoreType.DMA,
      ],
  )
  def spmm(src_hbm, dst_hbm, u_hbm, out_hbm, si0, si1, di0, di1, rb0, rb1,
           zb_v, acc_sh, semg0, semg1):
    c = lax.axis_index("c")
    s = lax.axis_index("s")
    tid = c * _NTILES + s
    zeros16 = jnp.zeros((16,), jnp.float32)

    def zero_zb(i, carry):
      r = i // (_CW // 16)
      col = (i % (_CW // 16)) * 16
      zb_v[r, pl.ds(col, 16)] = zeros16
      return carry
    lax.fori_loop(0, (_K // 2) * _CW // 16, zero_zb, 0)

    for p in range(ncb):
      for k in range(_RPT // (_K // 2)):
        pltpu.sync_copy(zb_v,
                        acc_sh.at[pl.ds(s * _RPT + k * (_K // 2), _K // 2)])
      plsc.subcore_barrier()

      pltpu.sync_copy(src_hbm.at[tid, 0], si0)
      pltpu.sync_copy(dst_hbm.at[tid, 0], di0)
      pltpu.async_copy(u_hbm.at[p].at[si0], rb0, semg0)

      def pair(jj, carry):
        j0 = 2 * jj
        pltpu.sync_copy(src_hbm.at[tid, j0 + 1], si1)
        pltpu.sync_copy(dst_hbm.at[tid, j0 + 1], di1)
        pltpu.async_copy(u_hbm.at[p].at[si1], rb1, semg1)

        pltpu.make_async_copy(u_hbm.at[p].at[si0], rb0, semg0).wait()
        pltpu.sync_copy(rb0, acc_sh.at[di0], add=True)

        @pl.when(j0 + 2 < _NCHUNK)
        def _():
          pltpu.sync_copy(src_hbm.at[tid, j0 + 2], si0)
          pltpu.sync_copy(dst_hbm.at[tid, j0 + 2], di0)
          pltpu.async_copy(u_hbm.at[p].at[si0], rb0, semg0)

        pltpu.make_async_copy(u_hbm.at[p].at[si1], rb1, semg1).wait()
        pltpu.sync_copy(rb1, acc_sh.at[di1], add=True)
        return carry
      lax.fori_loop(0, _NCHUNK // 2, pair, 0)

      plsc.subcore_barrier()
      pltpu.sync_copy(acc_sh.at[pl.ds(s * _RPT, _RPT)],
                      out_hbm.at[c, p, pl.ds(s * _RPT, _RPT)])
      if p + 1 < ncb:
        plsc.subcore_barrier()

  return spmm


_SC_SPMM = {ncb: _make_sc_spmm(ncb) for ncb in (1, 2, 4, 8)}


_BN = 1000
_GRID = _N // _BN


def _bfmm(a, b):
  return jnp.dot(a.astype(jnp.bfloat16), b.astype(jnp.bfloat16),
                 preferred_element_type=jnp.float32)


def _prep_body(x_ref, dega_ref, degb_ref, w1_ref, dis_ref, mu_ref):
  deg = dega_ref[...] + degb_ref[...] + 1.0
  dis = lax.rsqrt(deg)
  dis_ref[...] = jnp.broadcast_to(dis, (_BN, 128))
  mu_ref[0] = _bfmm(x_ref[...], w1_ref[...]) * dis


def _tc_prep(x, dega, degb, W1):
  return pl.pallas_call(
      _prep_body,
      grid=(_GRID,),
      in_specs=[
          pl.BlockSpec((_BN, 128), lambda i: (i, 0)),
          pl.BlockSpec((_BN, 1), lambda i: (i, 0)),
          pl.BlockSpec((_BN, 1), lambda i: (i, 0)),
          pl.BlockSpec((128, 128), lambda i: (0, 0)),
      ],
      out_specs=[
          pl.BlockSpec((_BN, 128), lambda i: (i, 0)),
          pl.BlockSpec((1, _BN, 128), lambda i: (0, i, 0)),
      ],
      out_shape=[
          jax.ShapeDtypeStruct((_N, 128), jnp.float32),
          jax.ShapeDtypeStruct((1, _N, 128), jnp.float32),
      ],
  )(x, dega, degb, W1)


def _make_layer_body(ncb_in, ncb_out):
  def body(s_ref, mu_ref, dis_ref, b_ref, w_ref, muo_ref):
    cols = [s_ref[0, p] + s_ref[1, p] + mu_ref[p] for p in range(ncb_in)]
    hcat = cols[0] if ncb_in == 1 else jnp.concatenate(cols, axis=1)
    dis = dis_ref[:, 0:1]
    h = jnp.maximum(hcat * dis + b_ref[...], 0.0)
    mun = _bfmm(h, w_ref[...]) * dis
    for p in range(ncb_out):
      muo_ref[p] = mun[:, p * _CW:(p + 1) * _CW]
  return body


def _tc_layer(s_p, mu, dis, b_prev, W_next, ncb_in, ncb_out):
  d_in = ncb_in * _CW
  d_out = ncb_out * _CW
  return pl.pallas_call(
      _make_layer_body(ncb_in, ncb_out),
      grid=(_GRID,),
      in_specs=[
          pl.BlockSpec((2, ncb_in, _BN, _CW), lambda i: (0, 0, i, 0)),
          pl.BlockSpec((ncb_in, _BN, _CW), lambda i: (0, i, 0)),
          pl.BlockSpec((_BN, 128), lambda i: (i, 0)),
          pl.BlockSpec((1, d_in), lambda i: (0, 0)),
          pl.BlockSpec((d_in, d_out), lambda i: (0, 0)),
      ],
      out_specs=pl.BlockSpec((ncb_out, _BN, _CW), lambda i: (0, i, 0)),
      out_shape=jax.ShapeDtypeStruct((ncb_out, _N, _CW), jnp.float32),
  )(s_p, mu, dis, b_prev.reshape(1, d_in), W_next)


def _final_body(s_ref, mu_ref, dis_ref, b_ref, batch_ref, out_ref,
                psum, cnts):
  i = pl.program_id(0)

  @pl.when(i == 0)
  def _():
    psum[...] = jnp.zeros_like(psum)
    cnts[...] = jnp.zeros_like(cnts)

  cols = [s_ref[0, p] + s_ref[1, p] + mu_ref[p] for p in range(8)]
  hcat = jnp.concatenate(cols, axis=1)
  dis = dis_ref[:, 0:1]
  h4 = jnp.maximum(hcat * dis + b_ref[...], 0.0)

  gid = lax.broadcasted_iota(jnp.int32, (_G, _BN), 0)
  bmat = (batch_ref[:, 0][None, :] == gid).astype(jnp.float32)
  psum[...] += jnp.dot(bmat, h4, preferred_element_type=jnp.float32,
                       precision=_PREC)
  cnts[:, 0:1] += jnp.sum(bmat, axis=1, keepdims=True)

  @pl.when(i == _GRID - 1)
  def _():
    out_ref[...] = psum[...] / jnp.maximum(cnts[:, 0:1], 1.0)


def _tc_final(s_p, mu, dis, b4, batch2d):
  return pl.pallas_call(
      _final_body,
      grid=(_GRID,),
      in_specs=[
          pl.BlockSpec((2, 8, _BN, _CW), lambda i: (0, 0, i, 0)),
          pl.BlockSpec((8, _BN, _CW), lambda i: (0, i, 0)),
          pl.BlockSpec((_BN, 128), lambda i: (i, 0)),
          pl.BlockSpec((1, 1024), lambda i: (0, 0)),
          pl.BlockSpec((_BN, 1), lambda i: (i, 0)),
      ],
      out_specs=pl.BlockSpec((_G, 1024), lambda i: (0, 0)),
      out_shape=jax.ShapeDtypeStruct((_G, 1024), jnp.float32),
      scratch_shapes=[
          pltpu.VMEM((_G, 1024), jnp.float32),
          pltpu.VMEM((_G, 128), jnp.float32),
      ],
  )(s_p, mu, dis, b4.reshape(1, 1024), batch2d)


def _head_body(pooled_ref, fcw1_ref, fcb1_ref, fcw2_ref, fcb2_ref, out_ref):
  o1 = _bfmm(pooled_ref[...], fcw1_ref[...]) + fcb1_ref[...]
  out_ref[...] = _bfmm(o1, fcw2_ref[...]) + fcb2_ref[...]


def _tc_head(pooled, fcW1, fcb1, fcW2, fcb2):
  return pl.pallas_call(
      _head_body,
      out_shape=jax.ShapeDtypeStruct((_G, 1), jnp.float32),
  )(pooled, fcW1, fcb1.reshape(1, 512), fcW2, fcb2.reshape(1, 1))


@jax.jit
def kernel(x, edge_index, batch, W1, b1, W2, b2, W3, b3, W4, b4,
           fcW1, fcb1, fcW2, fcb2):
  pad = _EPAD - _E
  src = jnp.concatenate(
      [edge_index[0], jnp.zeros((pad,), jnp.int32)]).reshape(
          _NW, _NCHUNK, _K)
  dst = jnp.concatenate(
      [edge_index[1], jnp.full((pad,), _NPAD - 1, jnp.int32)]).reshape(
          _NW, _NCHUNK, _K)

  degp = _sc_degree(dst)
  dega = degp[0, :_N, 0].reshape(_N, 1)
  degb = degp[1, :_N, 0].reshape(_N, 1)
  dis, mu = _tc_prep(x, dega, degb, W1)

  dims = [(1, 2, b1, W2), (2, 4, b2, W3), (4, 8, b3, W4)]
  for ncb_in, ncb_out, b_prev, W_next in dims:
    s_p = _SC_SPMM[ncb_in](src, dst, mu)
    mu = _tc_layer(s_p, mu, dis, b_prev, W_next, ncb_in, ncb_out)

  s_p = _SC_SPMM[8](src, dst, mu)
  pooled = _tc_final(s_p, mu, dis, b4, batch.reshape(_N, 1))
  out = _tc_head(pooled, fcW1, fcb1, fcW2, fcb2)
  return out


# speedup vs baseline: 1.0122x; 1.0122x over previous
import functools

import jax
import jax.numpy as jnp
from jax import lax
from jax.experimental import pallas as pl
from jax.experimental.pallas import tpu as pltpu
from jax.experimental.pallas import tpu_sc as plsc

_N = 10000
_E = 320000
_G = 16
_CW = 128
_NTILES = 16
_NCORES = 2
_NW = _NCORES * _NTILES
_K = 128
_NCHUNK = 80
_EPT = _NCHUNK * _K
_EPAD = _NW * _EPT
_RPT = 640
_NPAD = _RPT * _NTILES

_PREC = lax.Precision.HIGHEST


def _sc_mesh():
  return plsc.VectorSubcoreMesh(core_axis_name="c", subcore_axis_name="s")


@functools.partial(
    pl.kernel,
    out_type=jax.ShapeDtypeStruct((_NCORES, _NPAD, _CW), jnp.float32),
    mesh=_sc_mesh(),
    scratch_types=[
        pltpu.VMEM((_K,), jnp.int32),
        pltpu.VMEM((_K, _CW), jnp.float32),
        pltpu.VMEM((_K, _CW), jnp.float32),
        pltpu.VMEM_SHARED((_NPAD, _CW), jnp.float32),
    ],
)
def _sc_degree(dst_hbm, out_hbm, di_v, ones_v, zb_v, acc_sh):
  c = lax.axis_index("c")
  s = lax.axis_index("s")
  tid = c * _NTILES + s
  zeros16 = jnp.zeros((16,), jnp.float32)
  ones16 = jnp.ones((16,), jnp.float32)

  def fill(i, carry):
    r = i // (_CW // 16)
    col = (i % (_CW // 16)) * 16
    ones_v[r, pl.ds(col, 16)] = ones16
    zb_v[r, pl.ds(col, 16)] = zeros16
    return carry
  lax.fori_loop(0, _K * _CW // 16, fill, 0)

  for k in range(_RPT // _K):
    pltpu.sync_copy(zb_v, acc_sh.at[pl.ds(s * _RPT + k * _K, _K)])
  plsc.subcore_barrier()

  def chunk(j, carry):
    pltpu.sync_copy(dst_hbm.at[tid, j], di_v)
    pltpu.sync_copy(ones_v, acc_sh.at[di_v], add=True)
    return carry
  lax.fori_loop(0, _NCHUNK, chunk, 0)

  plsc.subcore_barrier()
  pltpu.sync_copy(acc_sh.at[pl.ds(s * _RPT, _RPT)],
                  out_hbm.at[c, pl.ds(s * _RPT, _RPT)])


def _make_sc_spmm(ncb):
  @functools.partial(
      pl.kernel,
      out_type=jax.ShapeDtypeStruct((_NCORES, ncb, _NPAD, _CW), jnp.float32),
      mesh=_sc_mesh(),
      scratch_types=[
          pltpu.VMEM((_NCHUNK, _K), jnp.int32),
          pltpu.VMEM((_K,), jnp.int32),
          pltpu.VMEM((_K,), jnp.int32),
          pltpu.VMEM((_K, _CW), jnp.float32),
          pltpu.VMEM((_K, _CW), jnp.float32),
          pltpu.VMEM_SHARED((_NPAD, _CW), jnp.float32),
          pltpu.SemaphoreType.DMA,
          pltpu.SemaphoreType.DMA,
          pltpu.SemaphoreType.DMA,
          pltpu.SemaphoreType.DMA,
      ],
  )
  def spmm(src_hbm, dst_hbm, zeros_hbm, u_hbm, out_hbm, si_v, di0, di1,
           rb0, rb1, acc_sh, semg0, semg1, semd0, semd1):
    c = lax.axis_index("c")
    s = lax.axis_index("s")
    tid = c * _NTILES + s

    pltpu.sync_copy(src_hbm.at[tid], si_v)

    for p in range(ncb):
      pltpu.sync_copy(zeros_hbm, acc_sh.at[pl.ds(s * _RPT, _RPT)])
      plsc.subcore_barrier()

      pltpu.sync_copy(dst_hbm.at[tid, 0], di0)
      pltpu.async_copy(u_hbm.at[p].at[si_v.at[0]], rb0, semg0)
      pltpu.async_copy(dst_hbm.at[tid, 1], di1, semd1)

      def pair(jj, carry):
        j0 = 2 * jj
        pltpu.async_copy(u_hbm.at[p].at[si_v.at[j0 + 1]], rb1, semg1)
        pltpu.make_async_copy(u_hbm.at[p].at[si_v.at[j0]], rb0,
                              semg0).wait()
        pltpu.sync_copy(rb0, acc_sh.at[di0], add=True)

        @pl.when(j0 + 2 < _NCHUNK)
        def _():
          pltpu.async_copy(dst_hbm.at[tid, j0 + 2], di0, semd0)
          pltpu.async_copy(u_hbm.at[p].at[si_v.at[j0 + 2]], rb0, semg0)

        pltpu.make_async_copy(dst_hbm.at[tid, 1], di1, semd1).wait()
        pltpu.make_async_copy(u_hbm.at[p].at[si_v.at[j0 + 1]], rb1,
                              semg1).wait()
        pltpu.sync_copy(rb1, acc_sh.at[di1], add=True)

        @pl.when(j0 + 3 < _NCHUNK)
        def _():
          pltpu.async_copy(dst_hbm.at[tid, j0 + 3], di1, semd1)

        @pl.when(j0 + 2 < _NCHUNK)
        def _():
          pltpu.make_async_copy(dst_hbm.at[tid, 0], di0, semd0).wait()
        return carry
      lax.fori_loop(0, _NCHUNK // 2, pair, 0)

      plsc.subcore_barrier()
      pltpu.sync_copy(acc_sh.at[pl.ds(s * _RPT, _RPT)],
                      out_hbm.at[c, p, pl.ds(s * _RPT, _RPT)])
      if p + 1 < ncb:
        plsc.subcore_barrier()

  return spmm


_SC_SPMM = {ncb: _make_sc_spmm(ncb) for ncb in (1, 2, 4, 8)}


_BN = 1000
_GRID = _N // _BN


def _bfmm(a, b):
  return jnp.dot(a.astype(jnp.bfloat16), b.astype(jnp.bfloat16),
                 preferred_element_type=jnp.float32)


def _prep_body(x_ref, dega_ref, degb_ref, w1_ref, dis_ref, mu_ref):
  deg = dega_ref[...] + degb_ref[...] + 1.0
  dis = lax.rsqrt(deg)
  dis_ref[...] = jnp.broadcast_to(dis, (_BN, 128))
  mu_ref[0] = _bfmm(x_ref[...], w1_ref[...]) * dis


def _tc_prep(x, dega, degb, W1):
  return pl.pallas_call(
      _prep_body,
      grid=(_GRID,),
      in_specs=[
          pl.BlockSpec((_BN, 128), lambda i: (i, 0)),
          pl.BlockSpec((_BN, 1), lambda i: (i, 0)),
          pl.BlockSpec((_BN, 1), lambda i: (i, 0)),
          pl.BlockSpec((128, 128), lambda i: (0, 0)),
      ],
      out_specs=[
          pl.BlockSpec((_BN, 128), lambda i: (i, 0)),
          pl.BlockSpec((1, _BN, 128), lambda i: (0, i, 0)),
      ],
      out_shape=[
          jax.ShapeDtypeStruct((_N, 128), jnp.float32),
          jax.ShapeDtypeStruct((1, _N, 128), jnp.float32),
      ],
  )(x, dega, degb, W1)


def _make_layer_body(ncb_in, ncb_out):
  def body(s_ref, mu_ref, dis_ref, b_ref, w_ref, muo_ref):
    cols = [s_ref[0, p] + s_ref[1, p] + mu_ref[p] for p in range(ncb_in)]
    hcat = cols[0] if ncb_in == 1 else jnp.concatenate(cols, axis=1)
    dis = dis_ref[:, 0:1]
    h = jnp.maximum(hcat * dis + b_ref[...], 0.0)
    mun = _bfmm(h, w_ref[...]) * dis
    for p in range(ncb_out):
      muo_ref[p] = mun[:, p * _CW:(p + 1) * _CW]
  return body


def _tc_layer(s_p, mu, dis, b_prev, W_next, ncb_in, ncb_out):
  d_in = ncb_in * _CW
  d_out = ncb_out * _CW
  return pl.pallas_call(
      _make_layer_body(ncb_in, ncb_out),
      grid=(_GRID,),
      in_specs=[
          pl.BlockSpec((2, ncb_in, _BN, _CW), lambda i: (0, 0, i, 0)),
          pl.BlockSpec((ncb_in, _BN, _CW), lambda i: (0, i, 0)),
          pl.BlockSpec((_BN, 128), lambda i: (i, 0)),
          pl.BlockSpec((1, d_in), lambda i: (0, 0)),
          pl.BlockSpec((d_in, d_out), lambda i: (0, 0)),
      ],
      out_specs=pl.BlockSpec((ncb_out, _BN, _CW), lambda i: (0, i, 0)),
      out_shape=jax.ShapeDtypeStruct((ncb_out, _N, _CW), jnp.float32),
  )(s_p, mu, dis, b_prev.reshape(1, d_in), W_next)


def _final_body(s_ref, mu_ref, dis_ref, b_ref, batch_ref, out_ref,
                psum, cnts):
  i = pl.program_id(0)

  @pl.when(i == 0)
  def _():
    psum[...] = jnp.zeros_like(psum)
    cnts[...] = jnp.zeros_like(cnts)

  cols = [s_ref[0, p] + s_ref[1, p] + mu_ref[p] for p in range(8)]
  hcat = jnp.concatenate(cols, axis=1)
  dis = dis_ref[:, 0:1]
  h4 = jnp.maximum(hcat * dis + b_ref[...], 0.0)

  gid = lax.broadcasted_iota(jnp.int32, (_G, _BN), 0)
  bmat = (batch_ref[:, 0][None, :] == gid).astype(jnp.float32)
  psum[...] += jnp.dot(bmat, h4, preferred_element_type=jnp.float32,
                       precision=_PREC)
  cnts[:, 0:1] += jnp.sum(bmat, axis=1, keepdims=True)

  @pl.when(i == _GRID - 1)
  def _():
    out_ref[...] = psum[...] / jnp.maximum(cnts[:, 0:1], 1.0)


def _tc_final(s_p, mu, dis, b4, batch2d):
  return pl.pallas_call(
      _final_body,
      grid=(_GRID,),
      in_specs=[
          pl.BlockSpec((2, 8, _BN, _CW), lambda i: (0, 0, i, 0)),
          pl.BlockSpec((8, _BN, _CW), lambda i: (0, i, 0)),
          pl.BlockSpec((_BN, 128), lambda i: (i, 0)),
          pl.BlockSpec((1, 1024), lambda i: (0, 0)),
          pl.BlockSpec((_BN, 1), lambda i: (i, 0)),
      ],
      out_specs=pl.BlockSpec((_G, 1024), lambda i: (0, 0)),
      out_shape=jax.ShapeDtypeStruct((_G, 1024), jnp.float32),
      scratch_shapes=[
          pltpu.VMEM((_G, 1024), jnp.float32),
          pltpu.VMEM((_G, 128), jnp.float32),
      ],
  )(s_p, mu, dis, b4.reshape(1, 1024), batch2d)


def _head_body(pooled_ref, fcw1_ref, fcb1_ref, fcw2_ref, fcb2_ref, out_ref):
  o1 = _bfmm(pooled_ref[...], fcw1_ref[...]) + fcb1_ref[...]
  out_ref[...] = _bfmm(o1, fcw2_ref[...]) + fcb2_ref[...]


def _tc_head(pooled, fcW1, fcb1, fcW2, fcb2):
  return pl.pallas_call(
      _head_body,
      out_shape=jax.ShapeDtypeStruct((_G, 1), jnp.float32),
  )(pooled, fcW1, fcb1.reshape(1, 512), fcW2, fcb2.reshape(1, 1))


@jax.jit
def kernel(x, edge_index, batch, W1, b1, W2, b2, W3, b3, W4, b4,
           fcW1, fcb1, fcW2, fcb2):
  pad = _EPAD - _E
  src = jnp.concatenate(
      [edge_index[0], jnp.zeros((pad,), jnp.int32)]).reshape(
          _NW, _NCHUNK, _K)
  dst = jnp.concatenate(
      [edge_index[1], jnp.full((pad,), _NPAD - 1, jnp.int32)]).reshape(
          _NW, _NCHUNK, _K)

  degp = _sc_degree(dst)
  dega = degp[0, :_N, 0].reshape(_N, 1)
  degb = degp[1, :_N, 0].reshape(_N, 1)
  dis, mu = _tc_prep(x, dega, degb, W1)

  zeros_rpt = jnp.zeros((_RPT, _CW), jnp.float32)
  dims = [(1, 2, b1, W2), (2, 4, b2, W3), (4, 8, b3, W4)]
  for ncb_in, ncb_out, b_prev, W_next in dims:
    s_p = _SC_SPMM[ncb_in](src, dst, zeros_rpt, mu)
    mu = _tc_layer(s_p, mu, dis, b_prev, W_next, ncb_in, ncb_out)

  s_p = _SC_SPMM[8](src, dst, zeros_rpt, mu)
  pooled = _tc_final(s_p, mu, dis, b4, batch.reshape(_N, 1))
  out = _tc_head(pooled, fcW1, fcb1, fcW2, fcb2)
  return out


# speedup vs baseline: 3.7547x; 3.7096x over previous
import functools

import jax
import jax.numpy as jnp
from jax import lax
from jax.experimental import pallas as pl
from jax.experimental.pallas import tpu as pltpu
from jax.experimental.pallas import tpu_sc as plsc

_N = 10000
_E = 320000
_G = 16
_CW = 128
_NTILES = 16
_NCORES = 2
_NW = _NCORES * _NTILES
_K = 128
_NCHUNK = 80
_EPT = _NCHUNK * _K
_EPAD = _NW * _EPT
_RPT = 640
_NPAD = _RPT * _NTILES

_PREC = lax.Precision.HIGHEST


def _sc_mesh():
  return plsc.VectorSubcoreMesh(core_axis_name="c", subcore_axis_name="s")


@functools.partial(
    pl.kernel,
    out_type=jax.ShapeDtypeStruct((_NCORES, _NPAD, _CW), jnp.float32),
    mesh=_sc_mesh(),
    scratch_types=[
        pltpu.VMEM((_K,), jnp.int32),
        pltpu.VMEM((_K, _CW), jnp.float32),
        pltpu.VMEM((_K, _CW), jnp.float32),
        pltpu.VMEM_SHARED((_NPAD, _CW), jnp.float32),
    ],
)
def _sc_degree(dst_hbm, out_hbm, di_v, ones_v, zb_v, acc_sh):
  c = lax.axis_index("c")
  s = lax.axis_index("s")
  tid = c * _NTILES + s
  zeros16 = jnp.zeros((16,), jnp.float32)
  ones16 = jnp.ones((16,), jnp.float32)

  def fill(i, carry):
    r = i // (_CW // 16)
    col = (i % (_CW // 16)) * 16
    ones_v[r, pl.ds(col, 16)] = ones16
    zb_v[r, pl.ds(col, 16)] = zeros16
    return carry
  lax.fori_loop(0, _K * _CW // 16, fill, 0)

  for k in range(_RPT // _K):
    pltpu.sync_copy(zb_v, acc_sh.at[pl.ds(s * _RPT + k * _K, _K)])
  plsc.subcore_barrier()

  def chunk(j, carry):
    pltpu.sync_copy(dst_hbm.at[tid, j], di_v)
    pltpu.sync_copy(ones_v, acc_sh.at[di_v], add=True)
    return carry
  lax.fori_loop(0, _NCHUNK, chunk, 0)

  plsc.subcore_barrier()
  pltpu.sync_copy(acc_sh.at[pl.ds(s * _RPT, _RPT)],
                  out_hbm.at[c, pl.ds(s * _RPT, _RPT)])


def _make_sc_spmm(ncb):
  @functools.partial(
      pl.kernel,
      out_type=jax.ShapeDtypeStruct((_NCORES, ncb, _NPAD, _CW), jnp.float32),
      mesh=_sc_mesh(),
      scratch_types=[
          pltpu.VMEM((_NCHUNK, _K), jnp.int32),
          pltpu.VMEM((_K,), jnp.int32),
          pltpu.VMEM((_K,), jnp.int32),
          pltpu.VMEM((_K, _CW), jnp.float32),
          pltpu.VMEM((_K, _CW), jnp.float32),
          pltpu.VMEM_SHARED((_NPAD, _CW), jnp.float32),
          pltpu.SemaphoreType.DMA,
          pltpu.SemaphoreType.DMA,
          pltpu.SemaphoreType.DMA,
          pltpu.SemaphoreType.DMA,
      ],
  )
  def spmm(src_hbm, dst_hbm, zeros_hbm, u_hbm, out_hbm, si_v, di0, di1,
           rb0, rb1, acc_sh, semg0, semg1, semd0, semd1):
    c = lax.axis_index("c")
    s = lax.axis_index("s")
    tid = c * _NTILES + s

    pltpu.sync_copy(src_hbm.at[tid], si_v)

    for p in range(ncb):
      pltpu.sync_copy(zeros_hbm, acc_sh.at[pl.ds(s * _RPT, _RPT)])
      plsc.subcore_barrier()

      pltpu.sync_copy(dst_hbm.at[tid, 0], di0)
      pltpu.async_copy(u_hbm.at[p].at[si_v.at[0]], rb0, semg0)
      pltpu.async_copy(dst_hbm.at[tid, 1], di1, semd1)

      def pair(jj, carry):
        j0 = 2 * jj
        pltpu.async_copy(u_hbm.at[p].at[si_v.at[j0 + 1]], rb1, semg1)
        pltpu.make_async_copy(u_hbm.at[p].at[si_v.at[j0]], rb0,
                              semg0).wait()
        pltpu.sync_copy(rb0, acc_sh.at[di0], add=True)

        @pl.when(j0 + 2 < _NCHUNK)
        def _():
          pltpu.async_copy(dst_hbm.at[tid, j0 + 2], di0, semd0)
          pltpu.async_copy(u_hbm.at[p].at[si_v.at[j0 + 2]], rb0, semg0)

        pltpu.make_async_copy(dst_hbm.at[tid, 1], di1, semd1).wait()
        pltpu.make_async_copy(u_hbm.at[p].at[si_v.at[j0 + 1]], rb1,
                              semg1).wait()
        pltpu.sync_copy(rb1, acc_sh.at[di1], add=True)

        @pl.when(j0 + 3 < _NCHUNK)
        def _():
          pltpu.async_copy(dst_hbm.at[tid, j0 + 3], di1, semd1)

        @pl.when(j0 + 2 < _NCHUNK)
        def _():
          pltpu.make_async_copy(dst_hbm.at[tid, 0], di0, semd0).wait()
        return carry
      lax.fori_loop(0, _NCHUNK // 2, pair, 0)

      plsc.subcore_barrier()
      pltpu.sync_copy(acc_sh.at[pl.ds(s * _RPT, _RPT)],
                      out_hbm.at[c, p, pl.ds(s * _RPT, _RPT)])
      if p + 1 < ncb:
        plsc.subcore_barrier()

  return spmm


_SC_SPMM = {ncb: _make_sc_spmm(ncb) for ncb in (1, 2, 4, 8)}


_BN = 1000
_GRID = _N // _BN


def _bfmm(a, b):
  return jnp.dot(a.astype(jnp.bfloat16), b.astype(jnp.bfloat16),
                 preferred_element_type=jnp.float32)


def _prep_body(x_ref, dega_ref, degb_ref, w1_ref, dis_ref, mu_ref):
  deg = dega_ref[...] + degb_ref[...] + 1.0
  dis = lax.rsqrt(deg)
  dis_ref[...] = jnp.broadcast_to(dis, (_BN, 128))
  mu_ref[0] = _bfmm(x_ref[...], w1_ref[...]) * dis


def _tc_prep(x, dega, degb, W1):
  return pl.pallas_call(
      _prep_body,
      grid=(_GRID,),
      in_specs=[
          pl.BlockSpec((_BN, 128), lambda i: (i, 0)),
          pl.BlockSpec((_BN, 1), lambda i: (i, 0)),
          pl.BlockSpec((_BN, 1), lambda i: (i, 0)),
          pl.BlockSpec((128, 128), lambda i: (0, 0)),
      ],
      out_specs=[
          pl.BlockSpec((_BN, 128), lambda i: (i, 0)),
          pl.BlockSpec((1, _BN, 128), lambda i: (0, i, 0)),
      ],
      out_shape=[
          jax.ShapeDtypeStruct((_N, 128), jnp.float32),
          jax.ShapeDtypeStruct((1, _N, 128), jnp.float32),
      ],
  )(x, dega, degb, W1)


def _make_layer_body(ncb_in, ncb_out):
  def body(s_ref, mu_ref, dis_ref, b_ref, w_ref, muo_ref):
    cols = [s_ref[0, p] + s_ref[1, p] + mu_ref[p] for p in range(ncb_in)]
    hcat = cols[0] if ncb_in == 1 else jnp.concatenate(cols, axis=1)
    dis = dis_ref[:, 0:1]
    h = jnp.maximum(hcat * dis + b_ref[...], 0.0)
    mun = _bfmm(h, w_ref[...]) * dis
    for p in range(ncb_out):
      muo_ref[p] = mun[:, p * _CW:(p + 1) * _CW]
  return body


def _tc_layer(s_p, mu, dis, b_prev, W_next, ncb_in, ncb_out):
  d_in = ncb_in * _CW
  d_out = ncb_out * _CW
  return pl.pallas_call(
      _make_layer_body(ncb_in, ncb_out),
      grid=(_GRID,),
      in_specs=[
          pl.BlockSpec((2, ncb_in, _BN, _CW), lambda i: (0, 0, i, 0)),
          pl.BlockSpec((ncb_in, _BN, _CW), lambda i: (0, i, 0)),
          pl.BlockSpec((_BN, 128), lambda i: (i, 0)),
          pl.BlockSpec((1, d_in), lambda i: (0, 0)),
          pl.BlockSpec((d_in, d_out), lambda i: (0, 0)),
      ],
      out_specs=pl.BlockSpec((ncb_out, _BN, _CW), lambda i: (0, i, 0)),
      out_shape=jax.ShapeDtypeStruct((ncb_out, _N, _CW), jnp.float32),
  )(s_p, mu, dis, b_prev.reshape(1, d_in), W_next)


def _final_body(s_ref, mu_ref, dis_ref, b_ref, batch_ref, out_ref,
                psum, cnts):
  i = pl.program_id(0)

  @pl.when(i == 0)
  def _():
    psum[...] = jnp.zeros_like(psum)
    cnts[...] = jnp.zeros_like(cnts)

  cols = [s_ref[0, p] + s_ref[1, p] + mu_ref[p] for p in range(8)]
  hcat = jnp.concatenate(cols, axis=1)
  dis = dis_ref[:, 0:1]
  h4 = jnp.maximum(hcat * dis + b_ref[...], 0.0)

  gid = lax.broadcasted_iota(jnp.int32, (_G, _BN), 0)
  bmat = (batch_ref[:, 0][None, :] == gid).astype(jnp.float32)
  psum[...] += jnp.dot(bmat, h4, preferred_element_type=jnp.float32,
                       precision=_PREC)
  cnts[:, 0:1] += jnp.sum(bmat, axis=1, keepdims=True)

  @pl.when(i == _GRID - 1)
  def _():
    out_ref[...] = psum[...] / jnp.maximum(cnts[:, 0:1], 1.0)


def _tc_final(s_p, mu, dis, b4, batch2d):
  return pl.pallas_call(
      _final_body,
      grid=(_GRID,),
      in_specs=[
          pl.BlockSpec((2, 8, _BN, _CW), lambda i: (0, 0, i, 0)),
          pl.BlockSpec((8, _BN, _CW), lambda i: (0, i, 0)),
          pl.BlockSpec((_BN, 128), lambda i: (i, 0)),
          pl.BlockSpec((1, 1024), lambda i: (0, 0)),
          pl.BlockSpec((_BN, 1), lambda i: (i, 0)),
      ],
      out_specs=pl.BlockSpec((_G, 1024), lambda i: (0, 0)),
      out_shape=jax.ShapeDtypeStruct((_G, 1024), jnp.float32),
      scratch_shapes=[
          pltpu.VMEM((_G, 1024), jnp.float32),
          pltpu.VMEM((_G, 128), jnp.float32),
      ],
  )(s_p, mu, dis, b4.reshape(1, 1024), batch2d)


def _head_body(pooled_ref, fcw1_ref, fcb1_ref, fcw2_ref, fcb2_ref, out_ref):
  o1 = _bfmm(pooled_ref[...], fcw1_ref[...]) + fcb1_ref[...]
  out_ref[...] = _bfmm(o1, fcw2_ref[...]) + fcb2_ref[...]


def _tc_head(pooled, fcW1, fcb1, fcW2, fcb2):
  return pl.pallas_call(
      _head_body,
      out_shape=jax.ShapeDtypeStruct((_G, 1), jnp.float32),
  )(pooled, fcW1, fcb1.reshape(1, 512), fcW2, fcb2.reshape(1, 1))


@jax.jit
def kernel(x, edge_index, batch, W1, b1, W2, b2, W3, b3, W4, b4,
           fcW1, fcb1, fcW2, fcb2):
  pad = _EPAD - _E
  pad_iota = jnp.arange(pad, dtype=jnp.int32)
  src = jnp.concatenate(
      [edge_index[0], pad_iota % _N]).reshape(_NW, _NCHUNK, _K)
  dst = jnp.concatenate(
      [edge_index[1], _N + pad_iota % (_NPAD - _N)]).reshape(
          _NW, _NCHUNK, _K)

  degp = _sc_degree(dst)
  dega = degp[0, :_N, 0].reshape(_N, 1)
  degb = degp[1, :_N, 0].reshape(_N, 1)
  dis, mu = _tc_prep(x, dega, degb, W1)

  zeros_rpt = jnp.zeros((_RPT, _CW), jnp.float32)
  dims = [(1, 2, b1, W2), (2, 4, b2, W3), (4, 8, b3, W4)]
  for ncb_in, ncb_out, b_prev, W_next in dims:
    s_p = _SC_SPMM[ncb_in](src, dst, zeros_rpt, mu)
    mu = _tc_layer(s_p, mu, dis, b_prev, W_next, ncb_in, ncb_out)

  s_p = _SC_SPMM[8](src, dst, zeros_rpt, mu)
  pooled = _tc_final(s_p, mu, dis, b4, batch.reshape(_N, 1))
  out = _tc_head(pooled, fcW1, fcb1, fcW2, fcb2)
  return out
